# baseline (speedup 1.0000x reference)
import functools
import jax
import jax.numpy as jnp
import numpy as np
from jax import lax
from jax.experimental import pallas as pl
from jax.experimental.pallas import tpu as pltpu
from jax.experimental.pallas import tpu_sc as plsc

N = 10000
K = 15
T = 512
KP = 16
NTILES = 32
SPT = 320
NLAST = N - (NTILES - 1) * SPT
CH = 8
NCH = SPT // CH
NCH_LAST = NLAST // CH
LANES = 16
TW = T // 2
HALF = T // 2


def _sum_pack_body(comps_ref, out_ref):
    c = comps_ref[...]
    s = c[:, 0, :] + c[:, 1, :] + c[:, 2, :] + c[:, 3, :]
    r = lax.bitcast_convert_type(s, jnp.uint32) + jnp.uint32(0x8000)
    ra = r[:, :HALF]
    rb = r[:, HALF:]
    word = (ra >> jnp.uint32(16)) | (rb & jnp.uint32(0xFFFF0000))
    out_ref[...] = lax.bitcast_convert_type(word, jnp.int32)


def _sum_pack(comps):
    bn = 2000
    grid = N // bn
    return pl.pallas_call(
        _sum_pack_body,
        grid=(grid,),
        in_specs=[pl.BlockSpec((bn, 4, T), lambda i: (i, 0, 0))],
        out_specs=pl.BlockSpec((bn, TW), lambda i: (i, 0)),
        out_shape=jax.ShapeDtypeStruct((N, TW), jnp.int32),
    )(comps)


def _dense_body(off_ref, tr_ref, amp_ref, ph_ref, per_ref, tv_ref, out_ref):
    tv = tv_ref[...]
    per = per_ref[...]
    amp = amp_ref[...]
    ph = ph_ref[...]
    off = off_ref[...]
    tr = tr_ref[...]
    ang = (2.0 * np.pi) * tv / per.reshape(5, 1)
    sinb = jnp.sin(ang)
    cosb = jnp.cos(ang)
    base = off + tr * tv
    out_ref[...] = (
        base
        + jnp.dot(amp * jnp.cos(ph), sinb, preferred_element_type=jnp.float32)
        + jnp.dot(amp * jnp.sin(ph), cosb, preferred_element_type=jnp.float32)
    )


def _dense_signals(off, tr, amp, ph, per, tv):
    bn = 2000
    grid = N // bn
    return pl.pallas_call(
        _dense_body,
        grid=(grid,),
        in_specs=[
            pl.BlockSpec((bn, 1), lambda i: (i, 0)),
            pl.BlockSpec((bn, 1), lambda i: (i, 0)),
            pl.BlockSpec((bn, 5), lambda i: (i, 0)),
            pl.BlockSpec((bn, 5), lambda i: (i, 0)),
            pl.BlockSpec((1, 5), lambda i: (0, 0)),
            pl.BlockSpec((1, T), lambda i: (0, 0)),
        ],
        out_specs=pl.BlockSpec((bn, T), lambda i: (i, 0)),
        out_shape=jax.ShapeDtypeStruct((N, T), jnp.float32),
    )(off, tr, amp, ph, per, tv)


def _sc_body(s_hbm, dense_hbm, idx_hbm, w_hbm, out_hbm,
             idx_v, w_v, rows_a, rows_b, acc_a, acc_b,
             sem_a, sem_b, dsem_a, dsem_b, osem_a, osem_b):
    nc = 2
    wid = lax.axis_index("s") * nc + lax.axis_index("c")
    base = wid * SPT
    last = wid == NTILES - 1

    @pl.when(last)
    def _():
        pltpu.sync_copy(idx_hbm.at[pl.ds(base * KP, NLAST * KP)],
                        idx_v.at[pl.ds(0, NLAST * KP)])
        pltpu.sync_copy(w_hbm.at[pl.ds(base, NLAST)],
                        w_v.at[pl.ds(0, NLAST)])

    @pl.when(jnp.logical_not(last))
    def _():
        pltpu.sync_copy(idx_hbm.at[pl.ds(base * KP, SPT * KP)], idx_v)
        pltpu.sync_copy(w_hbm.at[pl.ds(base, SPT)], w_v)

    nch = jnp.where(last, NCH_LAST, NCH)

    gdn = lax.GatherDimensionNumbers(
        offset_dims=(), collapsed_slice_dims=(0,), start_index_map=(0,))

    def splat(vec, k):
        idxs = jnp.full((LANES,), k, jnp.int32)
        return lax.gather(vec, idxs[:, None], dimension_numbers=gdn,
                          slice_sizes=(1,),
                          mode=lax.GatherScatterMode.PROMISE_IN_BOUNDS)

    def gather_h(c, rows_ref, sem):
        return pltpu.make_async_copy(
            s_hbm.at[idx_v.at[pl.ds(c * (CH * KP), CH * KP)]], rows_ref, sem)

    def dense_h(c, acc_ref, sem):
        return pltpu.make_async_copy(
            dense_hbm.at[pl.ds(base + c * CH, CH)], acc_ref, sem)

    def out_h(c, acc_ref, sem):
        return pltpu.make_async_copy(
            acc_ref, out_hbm.at[pl.ds(base + c * CH, CH)], sem)

    def compute(c, rows_ref, acc_ref, sem, dsem, osem):
        dense_h(c, acc_ref, dsem).wait()
        gather_h(c, rows_ref, sem).wait()

        def station(s, carry):
            w_vec = w_v[c * CH + s, :]
            accs = [None] * (TW // LANES)
            for k in range(KP):
                wk = plsc.bitcast(splat(w_vec, k), jnp.bfloat16)
                r = s * KP + k
                for cw in range(TW // LANES):
                    rb = plsc.bitcast(rows_ref[r, pl.ds(cw * LANES, LANES)],
                                      jnp.bfloat16)
                    p = wk * rb
                    accs[cw] = p if k == 0 else accs[cw] + p
            for cw in range(TW // LANES):
                lo, hi = plsc.unpack(accs[cw], format=plsc.PackFormat.INTERLEAVED)
                plsc.addupdate(acc_ref.at[s, pl.ds(cw * LANES, LANES)], lo)
                plsc.addupdate(acc_ref.at[s, pl.ds(HALF + cw * LANES, LANES)], hi)
            return carry

        lax.fori_loop(0, CH, station, 0)
        out_h(c, acc_ref, osem).start()

    gather_h(0, rows_a, sem_a).start()
    dense_h(0, acc_a, dsem_a).start()
    npair = nch // 2

    def pair(i, carry):
        c0 = i * 2
        gather_h(c0 + 1, rows_b, sem_b).start()
        compute(c0, rows_a, acc_a, sem_a, dsem_a, osem_a)

        @pl.when(i < npair - 1)
        def _():
            gather_h(c0 + 2, rows_a, sem_a).start()

        @pl.when(i > 0)
        def _():
            out_h(0, acc_b, osem_b).wait()
        dense_h(c0 + 1, acc_b, dsem_b).start()
        compute(c0 + 1, rows_b, acc_b, sem_b, dsem_b, osem_b)

        @pl.when(i < npair - 1)
        def _():
            out_h(0, acc_a, osem_a).wait()
            dense_h(c0 + 2, acc_a, dsem_a).start()

        return carry

    lax.fori_loop(0, npair, pair, 0)
    out_h(0, acc_a, osem_a).wait()
    out_h(0, acc_b, osem_b).wait()


def _sc_gather(s_tab, dense, idx_flat, w_flat):
    mesh = plsc.VectorSubcoreMesh(core_axis_name="c", subcore_axis_name="s")
    return pl.kernel(
        _sc_body,
        mesh=mesh,
        compiler_params=pltpu.CompilerParams(needs_layout_passes=False),
        out_type=jax.ShapeDtypeStruct((N, T), jnp.float32),
        scratch_types=[
            pltpu.VMEM((SPT * KP,), jnp.int32),
            pltpu.VMEM((SPT, KP), jnp.int32),
            pltpu.VMEM((CH * KP, TW), jnp.int32),
            pltpu.VMEM((CH * KP, TW), jnp.int32),
            pltpu.VMEM((CH, T), jnp.float32),
            pltpu.VMEM((CH, T), jnp.float32),
            pltpu.SemaphoreType.DMA,
            pltpu.SemaphoreType.DMA,
            pltpu.SemaphoreType.DMA,
            pltpu.SemaphoreType.DMA,
            pltpu.SemaphoreType.DMA,
            pltpu.SemaphoreType.DMA,
        ],
    )(s_tab, dense, idx_flat, w_flat)


@jax.jit
def kernel(time_vector, constant_offset, linear_trend, emd_seasonal_components,
           residual_amplitudes, residual_phases, residual_periods,
           emd_spatial_weights, local_spatial_weights,
           neighbor_indices, neighbor_weights, local_weights):
    f32 = jnp.float32
    amp_p = residual_amplitudes.astype(f32)
    ph_p = residual_phases.astype(f32)
    per_p = residual_periods.astype(f32).reshape(1, 5)
    tv = time_vector.astype(f32).reshape(1, T)
    off = constant_offset.astype(f32).reshape(N, 1)
    tr = linear_trend.astype(f32).reshape(N, 1)

    we = emd_spatial_weights.astype(f32)
    wl = local_spatial_weights.astype(f32)
    cw = wl[:, None] * local_weights.astype(f32) + we[:, None] * neighbor_weights.astype(f32)
    w16 = jnp.concatenate([cw, (1.0 - we - wl)[:, None]], axis=1)
    wbits = lax.bitcast_convert_type(
        w16.astype(jnp.bfloat16), jnp.uint16).astype(jnp.int32)
    w_tab = wbits | (wbits << 16)

    idx_flat = jnp.concatenate(
        [neighbor_indices.astype(jnp.int32),
         jnp.arange(N, dtype=jnp.int32)[:, None]], axis=1).reshape(-1)

    s_packed = _sum_pack(emd_seasonal_components.astype(f32))
    dense = _dense_signals(off, tr, amp_p, ph_p, per_p, tv)
    return _sc_gather(s_packed, dense, idx_flat, w_tab)
